# baseline (speedup 1.0000x reference)
import jax
import jax.numpy as jnp
from jax import lax
from jax.experimental import pallas as pl
from jax.experimental.pallas import tpu as pltpu
from jax.experimental.pallas import tpu_sc as plsc

B, L, V, D = 4096, 50, 100000, 300
TAIL0 = D - 128
DT = 48
NC, NS = 2, 16
NW = NC * NS
BPW = B // NW
LANES = 16


def _pool_body(batcht_hbm, piece_hbm, pooled_hbm, idx_s, rows0, rows1,
               out_v, sem0, sem1, *, offs):
    wid = lax.axis_index("s") * NC + lax.axis_index("c")
    base = wid * BPW
    pltpu.sync_copy(batcht_hbm.at[:, pl.ds(base, BPW)], idx_s)

    bufs = ((rows0, sem0), (rows1, sem1))

    def gather(r, buf, sem):
        return pltpu.make_async_copy(piece_hbm.at[idx_s.at[r]], buf, sem)

    gather(0, rows0, sem0).start()
    gather(1, rows1, sem1).start()

    zero = jnp.zeros((LANES,), jnp.float32)

    def zbody(e, carry):
        for j in range(len(offs)):
            out_v[e, pl.ds(LANES * j, LANES)] = zero
        return carry

    lax.fori_loop(0, BPW, zbody, 0)

    def accumulate(buf):
        def ebody(e, carry):
            for j, off in enumerate(offs):
                plsc.addupdate(out_v.at[e, pl.ds(LANES * j, LANES)],
                               buf[e, pl.ds(off, LANES)])
            return carry
        lax.fori_loop(0, BPW, ebody, 0)

    def pair(i, carry):
        r0 = i * 2
        for b in range(2):
            buf, sem = bufs[b]
            r = r0 + b
            gather(r, buf, sem).wait()
            accumulate(buf)
            nxt = r + 2

            @pl.when(nxt < L)
            def _():
                gather(nxt, buf, sem).start()
        return carry

    lax.fori_loop(0, L // 2, pair, 0)
    pltpu.sync_copy(out_v, pooled_hbm.at[pl.ds(base, BPW)])


def _make_pool(offs):
    import functools
    mesh = plsc.VectorSubcoreMesh(core_axis_name="c", subcore_axis_name="s")
    out_w = LANES * len(offs)
    return pl.kernel(
        functools.partial(_pool_body, offs=offs),
        mesh=mesh,
        out_type=jax.ShapeDtypeStruct((B, out_w), jnp.float32),
        scratch_types=[
            pltpu.VMEM((L, BPW), jnp.int32),
            pltpu.VMEM((BPW, 128), jnp.float32),
            pltpu.VMEM((BPW, 128), jnp.float32),
            pltpu.VMEM((BPW, out_w), jnp.float32),
            pltpu.SemaphoreType.DMA,
            pltpu.SemaphoreType.DMA,
        ],
    )


_FULL_OFFS = tuple(LANES * j for j in range(8))
_TAIL_OFFS = (80, 96, 112)


def _mlp_body(x0_ref, x1_ref, xb_ref, lens_ref, w0_ref, w1_ref, wb_ref,
              b1_ref, w2_ref, b2_ref, w3_ref, b3_ref, o_ref):
    recip = 1.0 / lens_ref[...].astype(jnp.float32)
    cdims = (((1,), (1,)), ((), ()))

    def dot(x, w):
        return lax.dot_general(x, w, cdims,
                               preferred_element_type=jnp.float32)

    h1 = (dot(x0_ref[...] * recip, w0_ref[...])
          + dot(x1_ref[...] * recip, w1_ref[...])
          + dot(xb_ref[...] * recip, wb_ref[...]))
    h1 = jnp.maximum(h1 + b1_ref[...], 0.0)
    h2 = dot(h1, w2_ref[...])
    h2 = jnp.maximum(h2 + b2_ref[...], 0.0)
    h3 = jnp.sum(h2 * w3_ref[...], axis=1, keepdims=True)
    o_ref[...] = h3 + b3_ref[0, 0]


def _mlp(p0, p1, pb, lens, W0, W1c, Wb, b1, W2, b2, W3, b3):
    BB = 512
    grid = (B // BB,)
    return pl.pallas_call(
        _mlp_body,
        grid=grid,
        in_specs=[
            pl.BlockSpec((BB, 128), lambda i: (i, 0)),
            pl.BlockSpec((BB, 128), lambda i: (i, 0)),
            pl.BlockSpec((BB, DT), lambda i: (i, 0)),
            pl.BlockSpec((BB, 1), lambda i: (i, 0)),
            pl.BlockSpec((150, 128), lambda i: (0, 0)),
            pl.BlockSpec((150, 128), lambda i: (0, 0)),
            pl.BlockSpec((150, DT), lambda i: (0, 0)),
            pl.BlockSpec((1, 150), lambda i: (0, 0)),
            pl.BlockSpec((150, 150), lambda i: (0, 0)),
            pl.BlockSpec((1, 150), lambda i: (0, 0)),
            pl.BlockSpec((1, 150), lambda i: (0, 0)),
            pl.BlockSpec(memory_space=pltpu.MemorySpace.SMEM),
        ],
        out_specs=pl.BlockSpec((BB, 1), lambda i: (i, 0)),
        out_shape=jax.ShapeDtypeStruct((B, 1), jnp.float32),
    )(p0, p1, pb, lens, W0, W1c, Wb, b1, W2, b2, W3, b3)


def kernel(batch, lens, table, W1, b1, W2, b2, W3, b3):
    batcht = batch.T
    piece0 = table[:, :128]
    piece1 = table[:, 128:256]
    tail = table[:, TAIL0:]

    p0 = _make_pool(_FULL_OFFS)(batcht, piece0)
    piece1, p0 = lax.optimization_barrier((piece1, p0))
    p1 = _make_pool(_FULL_OFFS)(batcht, piece1)
    tail, p1 = lax.optimization_barrier((tail, p1))
    pb = _make_pool(_TAIL_OFFS)(batcht, tail)

    Wb = jnp.pad(W1[:, 256:], ((0, 0), (4, 0)))
    lens2 = lens.reshape(B, 1)
    out = _mlp(p0, p1, pb, lens2, W1[:, :128], W1[:, 128:256], Wb,
               b1.reshape(1, 150), W2, b2.reshape(1, 150), W3,
               b3.reshape(1, 1))
    return out.reshape((B,))
